# Initial kernel scaffold; baseline (speedup 1.0000x reference)
import jax
import jax.numpy as jnp
from jax.experimental import pallas as pl


def kernel(x_cat, tables):
    raise NotImplementedError("write your pallas kernel here")


# speedup vs baseline: 1.3966x; 1.3966x over previous
import functools

import jax
import jax.numpy as jnp
from jax import lax
from jax.experimental import pallas as pl
from jax.experimental.pallas import tpu as pltpu
from jax.experimental.pallas import tpu_sc as plsc

N_FIELDS = 26
VOCAB = 100000
EMB_DIM = 32
B = 1024
L = 50
TOTAL = B * L * N_FIELDS

_info = plsc.get_sparse_core_info()
_NC, _NS, _L = _info.num_cores, _info.num_subcores, _info.num_lanes
_NW = _NC * _NS

ROWS_PER_W = TOTAL // _NW
CHUNK = 128
NCHUNK = ROWS_PER_W // CHUNK
_ADDS = CHUNK // _L


def _make_sc_gather():
    mesh = plsc.VectorSubcoreMesh(core_axis_name="c", subcore_axis_name="s")

    @functools.partial(
        pl.kernel,
        mesh=mesh,
        compiler_params=pltpu.CompilerParams(use_tc_tiling_on_sc=False),
        out_type=jax.ShapeDtypeStruct((TOTAL, EMB_DIM), jnp.float32),
        scratch_types=[
            pltpu.VMEM((ROWS_PER_W,), jnp.int32),
            pltpu.VMEM((ROWS_PER_W,), jnp.int32),
            pltpu.VMEM((2, CHUNK, EMB_DIM), jnp.float32),
            pltpu.SemaphoreType.DMA,
        ],
    )
    def k(tables_hbm, idx_hbm, off_hbm, out_hbm, idx_v, off_v, rows_v, sem):
        wid = lax.axis_index("s") * _NC + lax.axis_index("c")
        base = wid * ROWS_PER_W
        pltpu.sync_copy(idx_hbm.at[pl.ds(base, ROWS_PER_W)], idx_v)
        pltpu.sync_copy(off_hbm, off_v)

        def flatten_chunk(c):
            for j in range(_ADDS):
                s = c * CHUNK + j * _L
                idx_v[pl.ds(s, _L)] = idx_v[pl.ds(s, _L)] + off_v[pl.ds(s, _L)]

        def fire(c, buf):
            return pltpu.async_copy(
                tables_hbm.at[idx_v.at[pl.ds(c * CHUNK, CHUNK)]],
                rows_v.at[buf], sem)

        def drain(c, buf):
            pltpu.make_async_copy(
                tables_hbm.at[idx_v.at[pl.ds(c * CHUNK, CHUNK)]],
                rows_v.at[buf], sem).wait()
            pltpu.sync_copy(rows_v.at[buf],
                            out_hbm.at[pl.ds(base + c * CHUNK, CHUNK)])

        flatten_chunk(0)
        fire(0, 0)

        def body(c, carry):
            flatten_chunk(c)
            for b in range(2):

                @pl.when((c % 2) == b)
                def _phase(b=b):
                    fire(c, b)
                    drain(c - 1, 1 - b)

            return carry

        lax.fori_loop(1, NCHUNK, body, 0)
        drain(NCHUNK - 1, (NCHUNK - 1) % 2)

    return k


_sc_gather = _make_sc_gather()


def kernel(x_cat, tables):
    flat_idx = x_cat.reshape(TOTAL).astype(jnp.int32)
    flat_tables = tables.reshape(N_FIELDS * VOCAB, EMB_DIM)
    off = jnp.tile(jnp.arange(N_FIELDS, dtype=jnp.int32) * VOCAB,
                   ROWS_PER_W // N_FIELDS)
    out = _sc_gather(flat_tables, flat_idx, off)
    return out.reshape(B, L, N_FIELDS, EMB_DIM)
